# Initial kernel scaffold; baseline (speedup 1.0000x reference)
import jax
import jax.numpy as jnp
from jax.experimental import pallas as pl


def kernel(x, W, logit_bias, null_logit):
    raise NotImplementedError("write your pallas kernel here")


# speedup vs baseline: 2.1630x; 2.1630x over previous
import jax
import jax.numpy as jnp
from jax.experimental import pallas as pl
from jax.experimental.pallas import tpu as pltpu

_E = 64
_NULL = 64
_K = 8
_RHO = 0.5


def _gate_kernel(x_ref, w_ref, b_ref, n_ref, idx_ref, wgt_ref, nul_ref, aux_ref,
                 accP, accC, accS, *, t_blk, n_steps, n_tokens):
    step = pl.program_id(0)

    @pl.when(step == 0)
    def _init():
        accP[...] = jnp.zeros_like(accP)
        accC[...] = jnp.zeros_like(accC)
        accS[0] = 0.0
        accS[1] = 0.0

    x = x_ref[...]
    logits = jax.lax.dot_general(
        x, w_ref[...], (((1,), (1,)), ((), ())),
        preferred_element_type=jnp.float32) + b_ref[...]
    nl = n_ref[0, 0]

    m0 = jnp.max(logits, axis=1, keepdims=True)
    er = jnp.exp(logits - m0)
    zr = jnp.sum(er, axis=1, keepdims=True)
    accP[...] += jnp.sum(er / zr, axis=0, keepdims=True)

    M = jnp.maximum(m0, nl)
    e_all = jnp.exp(logits - M)
    en = jnp.exp(nl - M)
    Z = jnp.sum(e_all, axis=1, keepdims=True) + _NULL * en
    lse = M[:, 0] + jnp.log(Z[:, 0])
    accS[0] += jnp.sum(lse * lse)

    iota = jax.lax.broadcasted_iota(jnp.int32, logits.shape, 1)
    mask = jnp.ones(logits.shape, jnp.bool_)
    null_cnt = jnp.zeros((t_blk,), jnp.int32)
    neg = jnp.float32(-jnp.inf)
    idx_cols, val_cols, isn_cols = [], [], []
    for _ in range(_K):
        ml = jnp.where(mask, logits, neg)
        m = jnp.max(ml, axis=1)
        am = jnp.min(jnp.where(ml == m[:, None], iota, _E), axis=1)
        pick_real = m >= nl
        idx_cols.append(jnp.where(pick_real, am, _E + null_cnt))
        val_cols.append(jnp.where(pick_real, m, nl))
        isn_cols.append((~pick_real).astype(jnp.int32))
        null_cnt = null_cnt + (~pick_real).astype(jnp.int32)
        mask = mask & ~(pick_real[:, None] & (iota == am[:, None]))

    idx = jnp.stack(idx_cols, axis=1)
    vals = jnp.stack(val_cols, axis=1)
    isn = jnp.stack(isn_cols, axis=1)

    p_sel = jnp.exp(vals - M) / Z
    rw = jnp.where(isn != 0, 0.0, p_sel)
    ws = jnp.maximum(jnp.sum(rw, axis=1, keepdims=True), 1e-6)

    idx_ref[...] = idx
    wgt_ref[...] = rw / ws
    nul_ref[...] = isn

    accC[...] += jnp.sum((~mask).astype(jnp.float32), axis=0, keepdims=True)
    accS[1] += jnp.sum(null_cnt.astype(jnp.float32))

    @pl.when(step == n_steps - 1)
    def _fin():
        P = accP[0, :] / n_tokens
        counts = accC[0, :]
        total = jnp.maximum(jnp.sum(counts), 1e-6)
        L_bal = _E * jnp.sum((counts / total) * P)
        null_rate = accS[1] / (n_tokens * _K)
        L_null = (null_rate - _RHO) ** 2
        L_z = accS[0] / n_tokens
        aux = 0.02 * L_bal + 0.001 * L_z + 0.01 * L_null
        aux_ref[...] = jnp.full((1, 1), aux, jnp.float32)


def kernel(x, W, logit_bias, null_logit):
    B, T, D = x.shape
    N = B * T
    xf = x.reshape(N, D)
    b = logit_bias.reshape(1, _E).astype(jnp.float32)
    n = jnp.asarray(null_logit, jnp.float32).reshape(1, 1)

    t_blk = 1024
    n_steps = N // t_blk

    import functools
    idx, wgt, nul, aux = pl.pallas_call(
        functools.partial(_gate_kernel, t_blk=t_blk, n_steps=n_steps,
                          n_tokens=float(N)),
        grid=(n_steps,),
        in_specs=[
            pl.BlockSpec((t_blk, D), lambda i: (i, 0)),
            pl.BlockSpec((_E, D), lambda i: (0, 0)),
            pl.BlockSpec((1, _E), lambda i: (0, 0)),
            pl.BlockSpec((1, 1), lambda i: (0, 0)),
        ],
        out_specs=[
            pl.BlockSpec((t_blk, _K), lambda i: (i, 0)),
            pl.BlockSpec((t_blk, _K), lambda i: (i, 0)),
            pl.BlockSpec((t_blk, _K), lambda i: (i, 0)),
            pl.BlockSpec((1, 1), lambda i: (0, 0)),
        ],
        out_shape=[
            jax.ShapeDtypeStruct((N, _K), jnp.int32),
            jax.ShapeDtypeStruct((N, _K), jnp.float32),
            jax.ShapeDtypeStruct((N, _K), jnp.int32),
            jax.ShapeDtypeStruct((1, 1), jnp.float32),
        ],
        scratch_shapes=[
            pltpu.VMEM((1, _E), jnp.float32),
            pltpu.VMEM((1, _E), jnp.float32),
            pltpu.SMEM((2,), jnp.float32),
        ],
    )(xf, W, b, n)

    return (idx.reshape(B, T, _K),
            wgt.reshape(B, T, _K),
            nul.reshape(B, T, _K).astype(jnp.bool_),
            aux[0, 0])


# speedup vs baseline: 7.1984x; 3.3280x over previous
import functools

import jax
import jax.numpy as jnp
from jax.experimental import pallas as pl
from jax.experimental.pallas import tpu as pltpu

_E = 64
_NULL = 64
_K = 8
_RHO = 0.5
_IMIN = -2147483648
_FLIP = 0x7FFFFFFF
_LOW = 63


def _sortable(i):
    return jnp.where(i < 0, i ^ _FLIP, i)


def _gate_kernel(x_ref, w_ref, b_ref, n_ref, idx_ref, wgt_ref, nul_ref, aux_ref,
                 accP, accC, accS, *, n_steps, n_tokens):
    step = pl.program_id(0)

    @pl.when(step == 0)
    def _init():
        accP[...] = jnp.zeros_like(accP)
        accC[...] = jnp.zeros_like(accC)
        accS[0] = 0.0
        accS[1] = 0.0

    logits = jax.lax.dot_general(
        w_ref[...], x_ref[...], (((1,), (1,)), ((), ())),
        preferred_element_type=jnp.float32) + b_ref[...]
    nl = n_ref[0, 0]

    m0 = jnp.max(logits, axis=0, keepdims=True)
    er = jnp.exp(logits - m0)
    zr = jnp.sum(er, axis=0, keepdims=True)
    accP[...] += jnp.sum(er / zr, axis=1, keepdims=True)

    M = jnp.maximum(m0, nl)
    Z = zr * jnp.exp(m0 - M) + _NULL * jnp.exp(nl - M)
    lse = M + jnp.log(Z)
    accS[0] += jnp.sum(lse * lse)

    s = _sortable(jax.lax.bitcast_convert_type(logits, jnp.int32))
    iota_e = jax.lax.broadcasted_iota(jnp.int32, logits.shape, 0)
    nkey = _sortable(jax.lax.bitcast_convert_type(
        n_ref[...], jnp.int32))[0, 0]

    null_cnt = jnp.zeros((1, logits.shape[1]), jnp.int32)
    idx_rows, val_rows, isn_rows = [], [], []
    for _ in range(_K):
        smax = jnp.max(s, axis=0, keepdims=True)
        pick = smax >= nkey
        matches = s == smax
        am = jnp.min(jnp.where(matches, iota_e, _E), axis=0, keepdims=True)
        val = jax.lax.bitcast_convert_type(_sortable(smax), jnp.float32)
        idx_rows.append(jnp.where(pick, am, _E + null_cnt))
        val_rows.append(jnp.where(pick, val, nl))
        isn_rows.append(1 - pick.astype(jnp.int32))
        null_cnt = null_cnt + (1 - pick.astype(jnp.int32))
        s = jnp.where(matches & (iota_e == am) & pick, _IMIN, s)

    idx = jnp.concatenate(idx_rows, axis=0)
    vals = jnp.concatenate(val_rows, axis=0)
    isn = jnp.concatenate(isn_rows, axis=0)

    p_sel = jnp.exp(vals - M) / Z
    rw = jnp.where(isn != 0, 0.0, p_sel)
    ws = jnp.maximum(jnp.sum(rw, axis=0, keepdims=True), 1e-6)

    idx_ref[...] = idx
    wgt_ref[...] = rw / ws
    nul_ref[...] = isn

    accC[...] += jnp.sum((s == _IMIN).astype(jnp.float32), axis=1,
                         keepdims=True)
    accS[1] += jnp.sum(null_cnt.astype(jnp.float32))

    @pl.when(step == n_steps - 1)
    def _fin():
        P = accP[...] / n_tokens
        counts = accC[...]
        total = jnp.maximum(jnp.sum(counts), 1e-6)
        L_bal = _E * jnp.sum((counts / total) * P)
        null_rate = accS[1] / (n_tokens * _K)
        L_null = (null_rate - _RHO) ** 2
        L_z = accS[0] / n_tokens
        aux = 0.02 * L_bal + 0.001 * L_z + 0.01 * L_null
        aux_ref[...] = jnp.full((1, 1), aux, jnp.float32)


def kernel(x, W, logit_bias, null_logit):
    B, T, D = x.shape
    N = B * T
    xf = x.reshape(N, D)
    b = logit_bias.reshape(_E, 1).astype(jnp.float32)
    n = jnp.asarray(null_logit, jnp.float32).reshape(1, 1)

    t_blk = 1024
    n_steps = N // t_blk

    idx, wgt, nul, aux = pl.pallas_call(
        functools.partial(_gate_kernel, n_steps=n_steps, n_tokens=float(N)),
        grid=(n_steps,),
        in_specs=[
            pl.BlockSpec((t_blk, D), lambda i: (i, 0)),
            pl.BlockSpec((_E, D), lambda i: (0, 0)),
            pl.BlockSpec((_E, 1), lambda i: (0, 0)),
            pl.BlockSpec((1, 1), lambda i: (0, 0)),
        ],
        out_specs=[
            pl.BlockSpec((_K, t_blk), lambda i: (0, i)),
            pl.BlockSpec((_K, t_blk), lambda i: (0, i)),
            pl.BlockSpec((_K, t_blk), lambda i: (0, i)),
            pl.BlockSpec((1, 1), lambda i: (0, 0)),
        ],
        out_shape=[
            jax.ShapeDtypeStruct((_K, N), jnp.int32),
            jax.ShapeDtypeStruct((_K, N), jnp.float32),
            jax.ShapeDtypeStruct((_K, N), jnp.int32),
            jax.ShapeDtypeStruct((1, 1), jnp.float32),
        ],
        scratch_shapes=[
            pltpu.VMEM((_E, 1), jnp.float32),
            pltpu.VMEM((_E, 1), jnp.float32),
            pltpu.SMEM((2,), jnp.float32),
        ],
    )(xf, W, b, n)

    return (idx.T.reshape(B, T, _K),
            wgt.T.reshape(B, T, _K),
            nul.T.reshape(B, T, _K).astype(jnp.bool_),
            aux[0, 0])


# speedup vs baseline: 8.8794x; 1.2335x over previous
import functools

import jax
import jax.numpy as jnp
from jax.experimental import pallas as pl
from jax.experimental.pallas import tpu as pltpu

_E = 64
_NULL = 64
_K = 8
_RHO = 0.5
_IMIN = -2147483648
_FLIP = 0x7FFFFFFF
_LOW = 63


def _sortable(i):
    return jnp.where(i < 0, i ^ _FLIP, i)


def _gate_kernel(x_ref, w_ref, b_ref, n_ref, idx_ref, wgt_ref, nul_ref, aux_ref,
                 accP, accC, accS, *, n_steps, n_tokens):
    step = pl.program_id(0)

    @pl.when(step == 0)
    def _init():
        accP[...] = jnp.zeros_like(accP)
        accC[...] = jnp.zeros_like(accC)
        accS[0] = 0.0
        accS[1] = 0.0

    logits = jax.lax.dot_general(
        w_ref[...], x_ref[...], (((1,), (1,)), ((), ())),
        preferred_element_type=jnp.float32) + b_ref[...]
    nl = n_ref[0, 0]

    m0 = jnp.max(logits, axis=0, keepdims=True)
    er = jnp.exp(logits - m0)
    zr = jnp.sum(er, axis=0, keepdims=True)
    accP[...] += jnp.sum(er * (1.0 / zr), axis=1, keepdims=True)

    M = jnp.maximum(m0, nl)
    Z = zr * jnp.exp(m0 - M) + _NULL * jnp.exp(nl - M)
    lse = M + jnp.log(Z)
    accS[0] += jnp.sum(lse * lse)

    s = _sortable(jax.lax.bitcast_convert_type(logits, jnp.int32))
    iota_e = jax.lax.broadcasted_iota(jnp.int32, logits.shape, 0)
    nkey = _sortable(jax.lax.bitcast_convert_type(
        n_ref[...], jnp.int32))[0, 0]

    null_cnt = jnp.zeros((1, logits.shape[1]), jnp.int32)
    idx_rows, val_rows, isn_rows = [], [], []
    for _ in range(_K):
        smax = jnp.max(s, axis=0, keepdims=True)
        pick = smax >= nkey
        am = jnp.min(jnp.where(s == smax, iota_e, _E), axis=0, keepdims=True)
        val = jax.lax.bitcast_convert_type(_sortable(smax), jnp.float32)
        idx_rows.append(jnp.where(pick, am, _E + null_cnt))
        val_rows.append(jnp.where(pick, val, nl))
        isn_rows.append(1 - pick.astype(jnp.int32))
        null_cnt = null_cnt + (1 - pick.astype(jnp.int32))
        am2 = jnp.where(pick, am, -1)
        s = jnp.where(iota_e == am2, _IMIN, s)

    idx = jnp.concatenate(idx_rows, axis=0)
    vals = jnp.concatenate(val_rows, axis=0)
    isn = jnp.concatenate(isn_rows, axis=0)

    p_sel = jnp.exp(vals - M) * (1.0 / Z)
    rw = jnp.where(isn != 0, 0.0, p_sel)
    ws = jnp.maximum(jnp.sum(rw, axis=0, keepdims=True), 1e-6)

    idx_ref[...] = idx
    wgt_ref[...] = rw * (1.0 / ws)
    nul_ref[...] = isn

    accC[...] += jnp.sum((s == _IMIN).astype(jnp.float32), axis=1,
                         keepdims=True)
    accS[1] += jnp.sum(null_cnt.astype(jnp.float32))

    @pl.when(step == n_steps - 1)
    def _fin():
        P = accP[...] / n_tokens
        counts = accC[...]
        total = jnp.maximum(jnp.sum(counts), 1e-6)
        L_bal = _E * jnp.sum((counts / total) * P)
        null_rate = accS[1] / (n_tokens * _K)
        L_null = (null_rate - _RHO) ** 2
        L_z = accS[0] / n_tokens
        aux = 0.02 * L_bal + 0.001 * L_z + 0.01 * L_null
        aux_ref[...] = jnp.full((1, 1), aux, jnp.float32)


def kernel(x, W, logit_bias, null_logit):
    B, T, D = x.shape
    N = B * T
    xf = x.reshape(N, D)
    b = logit_bias.reshape(_E, 1).astype(jnp.float32)
    n = jnp.asarray(null_logit, jnp.float32).reshape(1, 1)

    t_blk = 2048
    n_steps = N // t_blk

    idx, wgt, nul, aux = pl.pallas_call(
        functools.partial(_gate_kernel, n_steps=n_steps, n_tokens=float(N)),
        grid=(n_steps,),
        in_specs=[
            pl.BlockSpec((t_blk, D), lambda i: (i, 0)),
            pl.BlockSpec((_E, D), lambda i: (0, 0)),
            pl.BlockSpec((_E, 1), lambda i: (0, 0)),
            pl.BlockSpec((1, 1), lambda i: (0, 0)),
        ],
        out_specs=[
            pl.BlockSpec((_K, t_blk), lambda i: (0, i)),
            pl.BlockSpec((_K, t_blk), lambda i: (0, i)),
            pl.BlockSpec((_K, t_blk), lambda i: (0, i)),
            pl.BlockSpec((1, 1), lambda i: (0, 0)),
        ],
        out_shape=[
            jax.ShapeDtypeStruct((_K, N), jnp.int32),
            jax.ShapeDtypeStruct((_K, N), jnp.float32),
            jax.ShapeDtypeStruct((_K, N), jnp.int32),
            jax.ShapeDtypeStruct((1, 1), jnp.float32),
        ],
        scratch_shapes=[
            pltpu.VMEM((_E, 1), jnp.float32),
            pltpu.VMEM((_E, 1), jnp.float32),
            pltpu.SMEM((2,), jnp.float32),
        ],
    )(xf, W, b, n)

    return (idx.T.reshape(B, T, _K),
            wgt.T.reshape(B, T, _K),
            nul.T.reshape(B, T, _K).astype(jnp.bool_),
            aux[0, 0])
